# baseline (speedup 1.0000x reference)
import functools

import jax
import jax.numpy as jnp
from jax.experimental import pallas as pl
from jax.experimental.pallas import tpu as pltpu


def _append_body(seq_ref, kb, nk, vb, nv, ok, ov, sem, *, bc, gd, c_rows, t_rows):
    seq = seq_ref[0]
    n_mid = t_rows // bc
    cb = bc * gd

    def _cp(src, s_off, dst, d_off, size):
        s_off = pl.multiple_of(s_off, gd)
        d_off = pl.multiple_of(d_off, gd)
        pltpu.make_async_copy(src.at[pl.ds(s_off, size)], dst.at[pl.ds(d_off, size)], sem).start()

    def head_chunk(i, n):
        base = i * cb
        _cp(kb, base, ok, base, cb)
        _cp(vb, base, ov, base, cb)
        return n + 2

    def mid_chunk(i, n):
        src = i * cb
        dst = (seq + i * bc) * gd
        _cp(nk, src, ok, dst, cb)
        _cp(nv, src, ov, dst, cb)
        return n + 2

    def tail_chunk(i, n):
        base = (seq + t_rows + i * bc) * gd
        _cp(kb, base, ok, base, cb)
        _cp(vb, base, ov, base, cb)
        return n + 2

    def head_row(i, n):
        base = ((seq // bc) * bc + i) * gd
        _cp(kb, base, ok, base, gd)
        _cp(vb, base, ov, base, gd)
        return n + 2

    def tail_row(i, n):
        base = (c_rows - ((c_rows - seq - t_rows) % bc) + i) * gd
        _cp(kb, base, ok, base, gd)
        _cp(vb, base, ov, base, gd)
        return n + 2

    n_chunks = jax.lax.fori_loop(0, seq // bc, head_chunk, 0)
    n_chunks = jax.lax.fori_loop(0, n_mid, mid_chunk, n_chunks)
    n_chunks = jax.lax.fori_loop(0, (c_rows - seq - t_rows) // bc, tail_chunk, n_chunks)
    n_rows = jax.lax.fori_loop(0, seq % bc, head_row, 0)
    n_rows = jax.lax.fori_loop(0, (c_rows - seq - t_rows) % bc, tail_row, n_rows)

    def wait_chunk(i, _):
        pltpu.make_async_copy(ok.at[pl.ds(0, cb)], ok.at[pl.ds(0, cb)], sem).wait()
        return 0

    def wait_row(i, _):
        pltpu.make_async_copy(ok.at[pl.ds(0, gd)], ok.at[pl.ds(0, gd)], sem).wait()
        return 0

    jax.lax.fori_loop(0, n_chunks, wait_chunk, 0)
    jax.lax.fori_loop(0, n_rows, wait_row, 0)


def kernel(keys_buffer, values_buffer, new_keys, new_values, sequence_length):
    C, G, D = keys_buffer.shape
    T = new_keys.shape[0]
    GD = G * D
    BC = 512

    kb1 = keys_buffer.reshape(C * GD)
    vb1 = values_buffer.reshape(C * GD)
    nk1 = new_keys.reshape(T * GD)
    nv1 = new_values.reshape(T * GD)
    seq = jnp.asarray(sequence_length, jnp.int32).reshape(1)

    hbm = pl.BlockSpec(memory_space=pltpu.MemorySpace.HBM)
    out_k, out_v = pl.pallas_call(
        functools.partial(_append_body, bc=BC, gd=GD, c_rows=C, t_rows=T),
        in_specs=[pl.BlockSpec(memory_space=pltpu.SMEM), hbm, hbm, hbm, hbm],
        out_specs=[hbm, hbm],
        out_shape=[
            jax.ShapeDtypeStruct((C * GD,), keys_buffer.dtype),
            jax.ShapeDtypeStruct((C * GD,), values_buffer.dtype),
        ],
        scratch_shapes=[pltpu.SemaphoreType.DMA],
    )(seq, kb1, nk1, vb1, nv1)

    new_seq = jnp.asarray(sequence_length + T, dtype=jnp.int32)
    return (new_seq, out_k.reshape(C, G, D), out_v.reshape(C, G, D))


# speedup vs baseline: 10.0198x; 10.0198x over previous
import functools

import jax
import jax.numpy as jnp
from jax.experimental import pallas as pl
from jax.experimental.pallas import tpu as pltpu


def _block_body(seq_ref, kb, nk, vb, nv, ok, ov, *, bc, tnb):
    i = pl.program_id(0)
    seq_b = seq_ref[0] // bc
    in_new = (i >= seq_b) & (i < seq_b + tnb)

    @pl.when(in_new)
    def _():
        ok[...] = nk[...]
        ov[...] = nv[...]

    @pl.when(jnp.logical_not(in_new))
    def _():
        ok[...] = kb[...]
        ov[...] = vb[...]


def _fast_kernel(kb2, vb2, nk2, nv2, seq, *, bc):
    C, GD = kb2.shape
    T = nk2.shape[0]
    nb = C // bc
    tnb = T // bc

    def buf_map(i, s):
        seq_b = s[0] // bc
        hi_b = seq_b + tnb
        interior = jnp.maximum(seq_b - 1, 0)
        return (jnp.where((i < seq_b) | (i >= hi_b), i, interior), 0)

    def new_map(i, s):
        seq_b = s[0] // bc
        return (jnp.clip(i - seq_b, 0, tnb - 1), 0)

    grid_spec = pltpu.PrefetchScalarGridSpec(
        num_scalar_prefetch=1,
        grid=(nb,),
        in_specs=[
            pl.BlockSpec((bc, GD), buf_map),
            pl.BlockSpec((bc, GD), new_map),
            pl.BlockSpec((bc, GD), buf_map),
            pl.BlockSpec((bc, GD), new_map),
        ],
        out_specs=[
            pl.BlockSpec((bc, GD), lambda i, s: (i, 0)),
            pl.BlockSpec((bc, GD), lambda i, s: (i, 0)),
        ],
    )

    return list(pl.pallas_call(
        functools.partial(_block_body, bc=bc, tnb=tnb),
        grid_spec=grid_spec,
        out_shape=[
            jax.ShapeDtypeStruct((C, GD), kb2.dtype),
            jax.ShapeDtypeStruct((C, GD), vb2.dtype),
        ],
        compiler_params=pltpu.CompilerParams(
            dimension_semantics=("arbitrary",),
        ),
    )(seq, kb2, nk2, vb2, nv2))


def _dma_body(seq_ref, kb, nk, vb, nv, ok, ov, sem, *, bc, gd, c_rows, t_rows):
    seq = seq_ref[0]
    cb = bc * gd

    def _cp(src, s_off, dst, d_off, size):
        s_off = pl.multiple_of(s_off, gd)
        d_off = pl.multiple_of(d_off, gd)
        pltpu.make_async_copy(src.at[pl.ds(s_off, size)], dst.at[pl.ds(d_off, size)], sem).start()

    def head_chunk(i, n):
        base = i * cb
        _cp(kb, base, ok, base, cb)
        _cp(vb, base, ov, base, cb)
        return n + 2

    def mid_chunk(i, n):
        src = i * cb
        dst = (seq + i * bc) * gd
        _cp(nk, src, ok, dst, cb)
        _cp(nv, src, ov, dst, cb)
        return n + 2

    def tail_chunk(i, n):
        base = (seq + t_rows + i * bc) * gd
        _cp(kb, base, ok, base, cb)
        _cp(vb, base, ov, base, cb)
        return n + 2

    def head_row(i, n):
        base = ((seq // bc) * bc + i) * gd
        _cp(kb, base, ok, base, gd)
        _cp(vb, base, ov, base, gd)
        return n + 2

    def tail_row(i, n):
        base = (c_rows - ((c_rows - seq - t_rows) % bc) + i) * gd
        _cp(kb, base, ok, base, gd)
        _cp(vb, base, ov, base, gd)
        return n + 2

    n_chunks = jax.lax.fori_loop(0, seq // bc, head_chunk, 0)
    n_chunks = jax.lax.fori_loop(0, t_rows // bc, mid_chunk, n_chunks)
    n_chunks = jax.lax.fori_loop(0, (c_rows - seq - t_rows) // bc, tail_chunk, n_chunks)
    n_rows = jax.lax.fori_loop(0, seq % bc, head_row, 0)
    n_rows = jax.lax.fori_loop(0, (c_rows - seq - t_rows) % bc, tail_row, n_rows)

    def wait_chunk(i, _):
        pltpu.make_async_copy(ok.at[pl.ds(0, cb)], ok.at[pl.ds(0, cb)], sem).wait()
        return 0

    def wait_row(i, _):
        pltpu.make_async_copy(ok.at[pl.ds(0, gd)], ok.at[pl.ds(0, gd)], sem).wait()
        return 0

    jax.lax.fori_loop(0, n_chunks, wait_chunk, 0)
    jax.lax.fori_loop(0, n_rows, wait_row, 0)


def _general_kernel(kb2, vb2, nk2, nv2, seq, *, bc):
    C, GD = kb2.shape
    T = nk2.shape[0]
    hbm = pl.BlockSpec(memory_space=pltpu.MemorySpace.HBM)
    out = pl.pallas_call(
        functools.partial(_dma_body, bc=bc, gd=GD, c_rows=C, t_rows=T),
        in_specs=[pl.BlockSpec(memory_space=pltpu.SMEM), hbm, hbm, hbm, hbm],
        out_specs=[hbm, hbm],
        out_shape=[
            jax.ShapeDtypeStruct((C * GD,), kb2.dtype),
            jax.ShapeDtypeStruct((C * GD,), vb2.dtype),
        ],
        scratch_shapes=[pltpu.SemaphoreType.DMA],
    )(seq, kb2.reshape(C * GD), nk2.reshape(T * GD), vb2.reshape(C * GD), nv2.reshape(T * GD))
    return [out[0].reshape(C, GD), out[1].reshape(C, GD)]


def kernel(keys_buffer, values_buffer, new_keys, new_values, sequence_length):
    C, G, D = keys_buffer.shape
    T = new_keys.shape[0]
    GD = G * D
    BC = 256

    kb2 = keys_buffer.reshape(C, GD)
    vb2 = values_buffer.reshape(C, GD)
    nk2 = new_keys.reshape(T, GD)
    nv2 = new_values.reshape(T, GD)
    seq_i32 = jnp.asarray(sequence_length, jnp.int32)
    seq = seq_i32.reshape(1)

    out_k, out_v = jax.lax.cond(
        seq_i32 % BC == 0,
        lambda: _fast_kernel(kb2, vb2, nk2, nv2, seq, bc=BC),
        lambda: _general_kernel(kb2, vb2, nk2, nv2, seq, bc=BC),
    )

    new_seq = jnp.asarray(sequence_length + T, dtype=jnp.int32)
    return (new_seq, out_k.reshape(C, G, D), out_v.reshape(C, G, D))


# speedup vs baseline: 47.7756x; 4.7681x over previous
import functools

import jax
import jax.numpy as jnp
from jax.experimental import pallas as pl
from jax.experimental.pallas import tpu as pltpu


def _block_body(seq_ref, kb, nk, vb, nv, ok, ov, *, tnb):
    i = pl.program_id(0)
    bc = ok.shape[0]
    seq_b = seq_ref[0] // bc
    in_new = (i >= seq_b) & (i < seq_b + tnb)

    @pl.when(in_new)
    def _():
        ok[...] = nk[...]
        ov[...] = nv[...]

    @pl.when(jnp.logical_not(in_new))
    def _():
        ok[...] = kb[...]
        ov[...] = vb[...]


def _fast_kernel(kb, vb, nk, nv, seq, *, bc):
    C, G, D = kb.shape
    T = nk.shape[0]
    nb = C // bc
    tnb = T // bc

    def buf_map(i, s):
        seq_b = s[0] // bc
        hi_b = seq_b + tnb
        interior = jnp.maximum(seq_b - 1, 0)
        return (jnp.where((i < seq_b) | (i >= hi_b), i, interior), 0, 0)

    def new_map(i, s):
        seq_b = s[0] // bc
        return (jnp.clip(i - seq_b, 0, tnb - 1), 0, 0)

    grid_spec = pltpu.PrefetchScalarGridSpec(
        num_scalar_prefetch=1,
        grid=(nb,),
        in_specs=[
            pl.BlockSpec((bc, G, D), buf_map),
            pl.BlockSpec((bc, G, D), new_map),
            pl.BlockSpec((bc, G, D), buf_map),
            pl.BlockSpec((bc, G, D), new_map),
        ],
        out_specs=[
            pl.BlockSpec((bc, G, D), lambda i, s: (i, 0, 0)),
            pl.BlockSpec((bc, G, D), lambda i, s: (i, 0, 0)),
        ],
    )

    return list(pl.pallas_call(
        functools.partial(_block_body, tnb=tnb),
        grid_spec=grid_spec,
        out_shape=[
            jax.ShapeDtypeStruct((C, G, D), kb.dtype),
            jax.ShapeDtypeStruct((C, G, D), vb.dtype),
        ],
        compiler_params=pltpu.CompilerParams(
            dimension_semantics=("arbitrary",),
        ),
    )(seq, kb, nk, vb, nv))


def _dma_body(seq_ref, kb, nk, vb, nv, ok, ov, sem, *, bc, gd, c_rows, t_rows):
    seq = seq_ref[0]
    cb = bc * gd

    def _cp(src, s_off, dst, d_off, size):
        s_off = pl.multiple_of(s_off, gd)
        d_off = pl.multiple_of(d_off, gd)
        pltpu.make_async_copy(src.at[pl.ds(s_off, size)], dst.at[pl.ds(d_off, size)], sem).start()

    def head_chunk(i, n):
        base = i * cb
        _cp(kb, base, ok, base, cb)
        _cp(vb, base, ov, base, cb)
        return n + 2

    def mid_chunk(i, n):
        src = i * cb
        dst = (seq + i * bc) * gd
        _cp(nk, src, ok, dst, cb)
        _cp(nv, src, ov, dst, cb)
        return n + 2

    def tail_chunk(i, n):
        base = (seq + t_rows + i * bc) * gd
        _cp(kb, base, ok, base, cb)
        _cp(vb, base, ov, base, cb)
        return n + 2

    def head_row(i, n):
        base = ((seq // bc) * bc + i) * gd
        _cp(kb, base, ok, base, gd)
        _cp(vb, base, ov, base, gd)
        return n + 2

    def tail_row(i, n):
        base = (c_rows - ((c_rows - seq - t_rows) % bc) + i) * gd
        _cp(kb, base, ok, base, gd)
        _cp(vb, base, ov, base, gd)
        return n + 2

    n_chunks = jax.lax.fori_loop(0, seq // bc, head_chunk, 0)
    n_chunks = jax.lax.fori_loop(0, t_rows // bc, mid_chunk, n_chunks)
    n_chunks = jax.lax.fori_loop(0, (c_rows - seq - t_rows) // bc, tail_chunk, n_chunks)
    n_rows = jax.lax.fori_loop(0, seq % bc, head_row, 0)
    n_rows = jax.lax.fori_loop(0, (c_rows - seq - t_rows) % bc, tail_row, n_rows)

    def wait_chunk(i, _):
        pltpu.make_async_copy(ok.at[pl.ds(0, cb)], ok.at[pl.ds(0, cb)], sem).wait()
        return 0

    def wait_row(i, _):
        pltpu.make_async_copy(ok.at[pl.ds(0, gd)], ok.at[pl.ds(0, gd)], sem).wait()
        return 0

    jax.lax.fori_loop(0, n_chunks, wait_chunk, 0)
    jax.lax.fori_loop(0, n_rows, wait_row, 0)


def _general_kernel(kb, vb, nk, nv, seq, *, bc):
    C, G, D = kb.shape
    T = nk.shape[0]
    GD = G * D
    hbm = pl.BlockSpec(memory_space=pltpu.MemorySpace.HBM)
    out = pl.pallas_call(
        functools.partial(_dma_body, bc=bc, gd=GD, c_rows=C, t_rows=T),
        in_specs=[pl.BlockSpec(memory_space=pltpu.SMEM), hbm, hbm, hbm, hbm],
        out_specs=[hbm, hbm],
        out_shape=[
            jax.ShapeDtypeStruct((C * GD,), kb.dtype),
            jax.ShapeDtypeStruct((C * GD,), vb.dtype),
        ],
        scratch_shapes=[pltpu.SemaphoreType.DMA],
    )(seq, kb.reshape(C * GD), nk.reshape(T * GD), vb.reshape(C * GD), nv.reshape(T * GD))
    return [out[0].reshape(C, G, D), out[1].reshape(C, G, D)]


def kernel(keys_buffer, values_buffer, new_keys, new_values, sequence_length):
    T = new_keys.shape[0]
    BC = 256

    seq_i32 = jnp.asarray(sequence_length, jnp.int32)
    seq = seq_i32.reshape(1)

    out_k, out_v = jax.lax.cond(
        seq_i32 % BC == 0,
        lambda: _fast_kernel(keys_buffer, values_buffer, new_keys, new_values, seq, bc=BC),
        lambda: _general_kernel(keys_buffer, values_buffer, new_keys, new_values, seq, bc=BC),
    )

    new_seq = jnp.asarray(sequence_length + T, dtype=jnp.int32)
    return (new_seq, out_k, out_v)
